# baseline (speedup 1.0000x reference)
import functools

import jax
import jax.numpy as jnp
from jax import lax
from jax.experimental import pallas as pl
from jax.experimental.pallas import tpu as pltpu
from jax.experimental.pallas import tpu_sc as plsc

_CHANNELS = 256
_GRID = 32
_N = 20000
_NW = 32
_BPW = 640
_CHUNK = 128
_NCHUNK = 2 * _BPW // _CHUNK
_NBUF = 4
_MAGIC = 12582912.0


@functools.partial(
    pl.kernel,
    mesh=plsc.VectorSubcoreMesh(core_axis_name="c", subcore_axis_name="s"),
    out_type=jax.ShapeDtypeStruct((2 * _N, 128), jnp.float32),
    scratch_types=[
        pltpu.VMEM((2 * _BPW,), jnp.float32),
        pltpu.VMEM((2 * _BPW,), jnp.float32),
        pltpu.VMEM((_NCHUNK, _CHUNK), jnp.int32),
    ] + [pltpu.VMEM((_CHUNK, 128), jnp.float32)] * _NBUF
    + [pltpu.SemaphoreType.DMA] * (2 * _NBUF),
)
def _pos_enc_sc(xy1_hbm, xy2_hbm, table_hbm, out_hbm, a_v, b_v, idx_v,
                buf0, buf1, buf2, buf3, gs0, gs1, gs2, gs3, ws0, ws1, ws2,
                ws3):
    bufs = (buf0, buf1, buf2, buf3)
    gsems = (gs0, gs1, gs2, gs3)
    wsems = (ws0, ws1, ws2, ws3)

    wid = lax.axis_index("s") * 2 + lax.axis_index("c")
    base = jnp.minimum(wid * _BPW, _N - _BPW)
    out_base = 2 * base

    cp_a = pltpu.async_copy(xy1_hbm.at[pl.ds(out_base, 2 * _BPW)], a_v, gs0)
    cp_b = pltpu.async_copy(xy2_hbm.at[pl.ds(out_base, 2 * _BPW)], b_v, gs1)
    cp_a.wait()
    cp_b.wait()

    lanes = lax.iota(jnp.int32, 16)
    yoff = (lanes % 2) * _GRID
    half = 0.5 * (_GRID - 1)

    gathers = [None] * _NCHUNK
    writes = [None] * _NCHUNK
    for c in range(_NCHUNK):
        for j in range(8):
            o = (c * 8 + j) * 16
            u = a_v[pl.ds(o, 16)] + b_v[pl.ds(o, 16)]
            r = (u * half + _MAGIC) - _MAGIC
            idx = jnp.clip(r, 0.0, _GRID - 1.0).astype(jnp.int32) + yoff
            idx_v[c, pl.ds(j * 16, 16)] = idx
        b = c % _NBUF
        if c >= _NBUF:
            writes[c - _NBUF].wait()
        gathers[c] = pltpu.async_copy(table_hbm.at[idx_v.at[c]], bufs[b],
                                      gsems[b])
        if c >= 1:
            gathers[c - 1].wait()
            writes[c - 1] = pltpu.async_copy(
                bufs[(c - 1) % _NBUF],
                out_hbm.at[pl.ds(out_base + (c - 1) * _CHUNK, _CHUNK)],
                wsems[(c - 1) % _NBUF])
    c = _NCHUNK - 1
    gathers[c].wait()
    writes[c] = pltpu.async_copy(
        bufs[c % _NBUF], out_hbm.at[pl.ds(out_base + c * _CHUNK, _CHUNK)],
        wsems[c % _NBUF])
    for c in range(_NCHUNK - _NBUF, _NCHUNK):
        writes[c].wait()


def kernel(boxes_norm, row_embed, col_embed):
    xy1 = boxes_norm[:, :2].reshape(-1)
    xy2 = boxes_norm[:, 2:].reshape(-1)
    table = jnp.concatenate([col_embed, row_embed], axis=0)
    out = _pos_enc_sc(xy1, xy2, table)
    return out.reshape(_N, _CHANNELS)[:, :, None, None]


# speedup vs baseline: 1.0011x; 1.0011x over previous
import functools

import jax
import jax.numpy as jnp
from jax import lax
from jax.experimental import pallas as pl
from jax.experimental.pallas import tpu as pltpu
from jax.experimental.pallas import tpu_sc as plsc

_CHANNELS = 256
_GRID = 32
_N = 20000
_NW = 32
_BPW = 640
_CHUNK = 128
_NCHUNK = 2 * _BPW // _CHUNK
_NBUF = 4
_MAGIC = 12582912.0


@functools.partial(
    pl.kernel,
    mesh=plsc.VectorSubcoreMesh(core_axis_name="c", subcore_axis_name="s"),
    out_type=jax.ShapeDtypeStruct((2 * _N, 128), jnp.float32),
    scratch_types=[
        pltpu.VMEM((2 * _BPW,), jnp.float32),
        pltpu.VMEM((2 * _BPW,), jnp.float32),
        pltpu.VMEM((_NCHUNK, _CHUNK), jnp.int32),
    ] + [pltpu.VMEM((_CHUNK, 128), jnp.float32)] * _NBUF
    + [pltpu.SemaphoreType.DMA] * (2 * _NBUF),
)
def _pos_enc_sc(xy1_hbm, xy2_hbm, table_hbm, out_hbm, a_v, b_v, idx_v,
                buf0, buf1, buf2, buf3, gs0, gs1, gs2, gs3, ws0, ws1, ws2,
                ws3):
    bufs = (buf0, buf1, buf2, buf3)
    gsems = (gs0, gs1, gs2, gs3)
    wsems = (ws0, ws1, ws2, ws3)

    wid = lax.axis_index("s") * 2 + lax.axis_index("c")
    base = jnp.minimum(wid * _BPW, _N - _BPW)
    out_base = 2 * base

    cp_a = pltpu.async_copy(xy1_hbm.at[pl.ds(out_base, 2 * _BPW)], a_v, gs0)
    cp_b = pltpu.async_copy(xy2_hbm.at[pl.ds(out_base, 2 * _BPW)], b_v, gs1)
    cp_a.wait()
    cp_b.wait()

    lanes = lax.iota(jnp.int32, 16)
    yoff = (lanes % 2) * _GRID
    half = 0.5 * (_GRID - 1)

    gathers = [None] * _NCHUNK
    for c in range(_NCHUNK):
        for j in range(8):
            o = (c * 8 + j) * 16
            u = a_v[pl.ds(o, 16)] + b_v[pl.ds(o, 16)]
            r = (u * half + _MAGIC) - _MAGIC
            idx = jnp.clip(r, 0.0, _GRID - 1.0).astype(jnp.int32) + yoff
            idx_v[c, pl.ds(j * 16, 16)] = idx
        b = c % _NBUF
        gathers[c] = pltpu.async_copy(table_hbm.at[idx_v.at[c]], bufs[b],
                                      gsems[b])
        if c >= 1:
            gathers[c - 1].wait()
            pltpu.sync_copy(
                bufs[(c - 1) % _NBUF],
                out_hbm.at[pl.ds(out_base + (c - 1) * _CHUNK, _CHUNK)])
    c = _NCHUNK - 1
    gathers[c].wait()
    pltpu.sync_copy(bufs[c % _NBUF],
                    out_hbm.at[pl.ds(out_base + c * _CHUNK, _CHUNK)])


def kernel(boxes_norm, row_embed, col_embed):
    xy1 = boxes_norm[:, :2].reshape(-1)
    xy2 = boxes_norm[:, 2:].reshape(-1)
    table = jnp.concatenate([col_embed, row_embed], axis=0)
    out = _pos_enc_sc(xy1, xy2, table)
    return out.reshape(_N, _CHANNELS)[:, :, None, None]


# speedup vs baseline: 2.0563x; 2.0540x over previous
import functools

import jax
import jax.numpy as jnp
from jax import lax
from jax.experimental import pallas as pl
from jax.experimental.pallas import tpu as pltpu
from jax.experimental.pallas import tpu_sc as plsc

_CHANNELS = 256
_GRID = 32
_N = 20000
_NW = 32
_BPW = 640
_CHUNK = 128
_NCHUNK = _BPW // _CHUNK
_MAGIC = 12582912.0


@functools.partial(
    pl.kernel,
    mesh=plsc.VectorSubcoreMesh(core_axis_name="c", subcore_axis_name="s"),
    out_type=jax.ShapeDtypeStruct((_N, _CHANNELS), jnp.float32),
    scratch_types=[
        pltpu.VMEM((4 * _BPW,), jnp.float32),
        pltpu.VMEM((_NCHUNK, _CHUNK), jnp.int32),
        pltpu.VMEM((_CHUNK, _CHANNELS), jnp.float32),
        pltpu.VMEM((_CHUNK, _CHANNELS), jnp.float32),
        pltpu.SemaphoreType.DMA,
        pltpu.SemaphoreType.DMA,
    ],
)
def _pos_enc_sc(boxes_hbm, sup_hbm, out_hbm, boxes_v, idx_v, buf0, buf1,
                sem0, sem1):
    bufs = (buf0, buf1)
    sems = (sem0, sem1)

    wid = lax.axis_index("s") * 2 + lax.axis_index("c")
    base = jnp.minimum(wid * _BPW, _N - _BPW)

    for i in range(4):
        pltpu.sync_copy(boxes_hbm.at[pl.ds(i * _N + base, _BPW)],
                        boxes_v.at[pl.ds(i * _BPW, _BPW)])

    half = 0.5 * (_GRID - 1)
    gathers = [None] * _NCHUNK
    for c in range(_NCHUNK):
        for j in range(8):
            o = (c * 8 + j) * 16
            sx = boxes_v[pl.ds(o, 16)] + boxes_v[pl.ds(2 * _BPW + o, 16)]
            sy = boxes_v[pl.ds(_BPW + o, 16)] + boxes_v[pl.ds(3 * _BPW + o, 16)]
            rx = (sx * half + _MAGIC) - _MAGIC
            ry = (sy * half + _MAGIC) - _MAGIC
            cxi = jnp.clip(rx, 0.0, _GRID - 1.0).astype(jnp.int32)
            cyi = jnp.clip(ry, 0.0, _GRID - 1.0).astype(jnp.int32)
            idx_v[c, pl.ds(j * 16, 16)] = cxi * _GRID + cyi
        gathers[c] = pltpu.async_copy(sup_hbm.at[idx_v.at[c]], bufs[c % 2],
                                      sems[c % 2])
        if c >= 1:
            gathers[c - 1].wait()
            pltpu.sync_copy(bufs[(c - 1) % 2],
                            out_hbm.at[pl.ds(base + (c - 1) * _CHUNK, _CHUNK)])
    c = _NCHUNK - 1
    gathers[c].wait()
    pltpu.sync_copy(bufs[c % 2],
                    out_hbm.at[pl.ds(base + c * _CHUNK, _CHUNK)])


def kernel(boxes_norm, row_embed, col_embed):
    boxes_t = boxes_norm.T.reshape(-1)
    sup = jnp.concatenate(
        [jnp.broadcast_to(col_embed[:, None, :], (_GRID, _GRID, 128)),
         jnp.broadcast_to(row_embed[None, :, :], (_GRID, _GRID, 128))],
        axis=-1).reshape(_GRID * _GRID, _CHANNELS)
    out = _pos_enc_sc(boxes_t, sup)
    return out[:, :, None, None]
